# baseline (speedup 1.0000x reference)
import jax
import jax.numpy as jnp
from jax.experimental import pallas as pl
from jax.experimental.pallas import tpu as pltpu

B = 4096
D = 64
NK = 10
KTH = 7
BIG = 1000000000.0
ROWS1 = 512
ROWS2 = 256
MED_RANK = B // 2 + (B % 2)


def _knn_kernel(inp_ref, rows_ref, dist_ref, kth_lane_ref, colsel_ref,
                sigma_ref, kth_scr, hits_scr):
    i = pl.program_id(0)
    n = pl.num_programs(0)
    allx = inp_ref[...]
    rows = rows_ref[...]
    r = rows.shape[0]

    gram = jax.lax.dot_general(rows, allx, (((1,), (1,)), ((), ())),
                               preferred_element_type=jnp.float32)
    rsq = jnp.sum(rows * rows, axis=1, keepdims=True)
    ones = jnp.ones((1, D), jnp.float32)
    csq = jax.lax.dot_general(ones, allx * allx, (((1,), (1,)), ((), ())),
                              preferred_element_type=jnp.float32,
                              precision=jax.lax.Precision.HIGHEST)
    d2raw = jnp.maximum(rsq + csq - 2.0 * gram, 0.0)

    row_g = i * r + jax.lax.broadcasted_iota(jnp.int32, (r, B), 0)
    col_i = jax.lax.broadcasted_iota(jnp.int32, (r, B), 1)
    d2 = jnp.where(row_g == col_i, BIG, d2raw)
    dist_ref[...] = d2.astype(jnp.bfloat16)

    c = B // 8
    a = [d2[:, j * c:(j + 1) * c] for j in range(8)]
    s = [jnp.minimum(a[2 * j], a[2 * j + 1]) for j in range(4)]
    l = [jnp.maximum(a[2 * j], a[2 * j + 1]) for j in range(4)]

    def _sort4(s1, l1, s2, l2):
        m_ = jnp.minimum(s1, s2)
        mm = jnp.maximum(s1, s2)
        n_ = jnp.minimum(l1, l2)
        return (m_, jnp.minimum(mm, n_), jnp.maximum(mm, n_),
                jnp.maximum(l1, l2))

    x1, x2, x3, x4 = _sort4(s[0], l[0], s[1], l[1])
    y1, y2, y3, y4 = _sort4(s[2], l[2], s[3], l[3])
    z1 = jnp.minimum(x1, y1)
    za = jnp.maximum(x1, y1)
    zb = jnp.minimum(x2, y2)
    z2 = jnp.minimum(za, zb)
    z3 = jnp.minimum(jnp.maximum(za, zb), jnp.minimum(x3, y3))
    z4 = jnp.minimum(
        jnp.minimum(jnp.minimum(x4, y4),
                    jnp.maximum(x3, y1)),
        jnp.minimum(jnp.maximum(x2, y2), jnp.maximum(x1, y3)))
    cand = jnp.concatenate([z1, z2, z3], axis=1)

    m = jnp.min(cand, axis=1, keepdims=True)
    kth = m
    for k in range(1, NK):
        m = jnp.min(jnp.where(cand <= m, BIG, cand), axis=1, keepdims=True)
        if k == KTH:
            kth = m
    t10 = m

    bad = jnp.any(z4 <= t10)

    kth_scr[...] = kth
    anyhit = jnp.any(d2 <= t10, axis=0, keepdims=True)
    hits_scr[...] = jnp.where(anyhit, 0.5, 0.0)

    @pl.when(bad)
    def _fallback():
        mm = jnp.min(d2, axis=1, keepdims=True)
        kk = mm
        for k in range(1, NK):
            mm = jnp.min(jnp.where(d2 <= mm, BIG, d2), axis=1,
                         keepdims=True)
            if k == KTH:
                kk = mm
        kth_scr[...] = kk
        ah = jnp.any(d2 <= mm, axis=0, keepdims=True)
        hits_scr[...] = jnp.where(ah, 0.5, 0.0)

    kth = kth_scr[...]
    hits = hits_scr[...]

    lr = jax.lax.broadcasted_iota(jnp.int32, (r, r), 0)
    lc = jax.lax.broadcasted_iota(jnp.int32, (r, r), 1)
    kl = jnp.sum(jnp.where(lr == lc, kth, 0.0), axis=0, keepdims=True)
    kth_lane_ref[:, pl.ds(i * r, r)] = kl

    @pl.when(i == 0)
    def _init():
        colsel_ref[...] = jnp.zeros_like(colsel_ref)

    colsel_ref[...] = jnp.maximum(colsel_ref[...], hits)

    @pl.when(i == n - 1)
    def _sigma():
        x = kth_lane_ref[...]

        def body(_, lohi):
            lo, hi = lohi
            mid = lo + ((hi - lo) >> 1)
            t = jax.lax.bitcast_convert_type(
                jnp.full((1, B), mid, jnp.int32), jnp.float32)
            cnt = jnp.sum((x <= t).astype(jnp.float32))
            return (jnp.where(cnt >= float(MED_RANK), lo, mid + 1),
                    jnp.where(cnt >= float(MED_RANK), mid, hi))

        lo, _ = jax.lax.fori_loop(
            0, 31, body, (jnp.int32(0), jnp.int32(0x7F800000)))
        med_d2 = jax.lax.bitcast_convert_type(lo, jnp.float32)
        sigma_ref[...] = jnp.broadcast_to(jnp.sqrt(jnp.sqrt(med_d2)), (1, 1))


def _aff_kernel(sigma_ref, colsel_ref, dist_ref, out_ref):
    i = pl.program_id(0)
    sigma = sigma_ref[0, 0]
    neg_inv = -1.0 / (2.0 * sigma * sigma)

    d2 = dist_ref[...].astype(jnp.float32)
    r = d2.shape[0]
    e = jnp.exp(jnp.sqrt(d2) * neg_inv)

    cs_lane = colsel_ref[...]
    cs_chunk = colsel_ref[:, pl.ds(i * r, r)]
    lr = jax.lax.broadcasted_iota(jnp.int32, (r, r), 0)
    lc = jax.lax.broadcasted_iota(jnp.int32, (r, r), 1)
    cs_row = jnp.sum(jnp.where(lr == lc, cs_chunk, 0.0), axis=1,
                     keepdims=True)
    out_ref[...] = e * (cs_row + cs_lane)


def kernel(inp):
    n1 = B // ROWS1
    dist, kth_lane, colsel, sigma = pl.pallas_call(
        _knn_kernel,
        grid=(n1,),
        in_specs=[
            pl.BlockSpec((B, D), lambda i: (0, 0)),
            pl.BlockSpec((ROWS1, D), lambda i: (i, 0)),
        ],
        out_specs=[
            pl.BlockSpec((ROWS1, B), lambda i: (i, 0)),
            pl.BlockSpec((1, B), lambda i: (0, 0)),
            pl.BlockSpec((1, B), lambda i: (0, 0)),
            pl.BlockSpec((1, 1), lambda i: (0, 0)),
        ],
        out_shape=[
            jax.ShapeDtypeStruct((B, B), jnp.bfloat16),
            jax.ShapeDtypeStruct((1, B), jnp.float32),
            jax.ShapeDtypeStruct((1, B), jnp.float32),
            jax.ShapeDtypeStruct((1, 1), jnp.float32),
        ],
        scratch_shapes=[
            pltpu.VMEM((ROWS1, 1), jnp.float32),
            pltpu.VMEM((1, B), jnp.float32),
        ],
        compiler_params=pltpu.CompilerParams(
            dimension_semantics=("arbitrary",)),
    )(inp, inp)

    n2 = B // ROWS2
    sym = pl.pallas_call(
        _aff_kernel,
        grid=(n2,),
        in_specs=[
            pl.BlockSpec((1, 1), lambda i: (0, 0)),
            pl.BlockSpec((1, B), lambda i: (0, 0)),
            pl.BlockSpec((ROWS2, B), lambda i: (i, 0)),
        ],
        out_specs=pl.BlockSpec((ROWS2, B), lambda i: (i, 0)),
        out_shape=jax.ShapeDtypeStruct((B, B), jnp.float32),
        compiler_params=pltpu.CompilerParams(
            dimension_semantics=("parallel",)),
    )(sigma, colsel, dist)
    return sym


# speedup vs baseline: 1.0168x; 1.0168x over previous
import jax
import jax.numpy as jnp
from jax.experimental import pallas as pl
from jax.experimental.pallas import tpu as pltpu

B = 4096
D = 64
NK = 10
KTH = 7
BIG = 1000000000.0
ROWS1 = 512
ROWS2 = 512
MED_RANK = B // 2 + (B % 2)


def _knn_kernel(inp_ref, rows_ref, dist_ref, kth_lane_ref, colsel_ref,
                sigma_ref, kth_scr, hits_scr):
    i = pl.program_id(0)
    n = pl.num_programs(0)
    allx = inp_ref[...]
    rows = rows_ref[...]
    r = rows.shape[0]

    gram = jax.lax.dot_general(rows, allx, (((1,), (1,)), ((), ())),
                               preferred_element_type=jnp.float32)
    rsq = jnp.sum(rows * rows, axis=1, keepdims=True)
    ones = jnp.ones((1, D), jnp.float32)
    csq = jax.lax.dot_general(ones, allx * allx, (((1,), (1,)), ((), ())),
                              preferred_element_type=jnp.float32,
                              precision=jax.lax.Precision.HIGHEST)
    d2raw = jnp.maximum(rsq + csq - 2.0 * gram, 0.0)

    row_g = i * r + jax.lax.broadcasted_iota(jnp.int32, (r, B), 0)
    col_i = jax.lax.broadcasted_iota(jnp.int32, (r, B), 1)
    d2 = jnp.where(row_g == col_i, BIG, d2raw)
    dist_ref[...] = d2.astype(jnp.bfloat16)

    c = B // 8
    a = [d2[:, j * c:(j + 1) * c] for j in range(8)]
    s = [jnp.minimum(a[2 * j], a[2 * j + 1]) for j in range(4)]
    l = [jnp.maximum(a[2 * j], a[2 * j + 1]) for j in range(4)]

    def _sort4(s1, l1, s2, l2):
        m_ = jnp.minimum(s1, s2)
        mm = jnp.maximum(s1, s2)
        n_ = jnp.minimum(l1, l2)
        return (m_, jnp.minimum(mm, n_), jnp.maximum(mm, n_),
                jnp.maximum(l1, l2))

    x1, x2, x3, x4 = _sort4(s[0], l[0], s[1], l[1])
    y1, y2, y3, y4 = _sort4(s[2], l[2], s[3], l[3])
    z1 = jnp.minimum(x1, y1)
    za = jnp.maximum(x1, y1)
    zb = jnp.minimum(x2, y2)
    z2 = jnp.minimum(za, zb)
    z3 = jnp.minimum(jnp.maximum(za, zb), jnp.minimum(x3, y3))
    z4 = jnp.minimum(
        jnp.minimum(jnp.minimum(x4, y4),
                    jnp.maximum(x3, y1)),
        jnp.minimum(jnp.maximum(x2, y2), jnp.maximum(x1, y3)))
    cand = jnp.concatenate([z1, z2, z3], axis=1)

    m = jnp.min(cand, axis=1, keepdims=True)
    kth = m
    for k in range(1, NK):
        m = jnp.min(jnp.where(cand <= m, BIG, cand), axis=1, keepdims=True)
        if k == KTH:
            kth = m
    t10 = m

    bad = jnp.any(z4 <= t10)

    kth_scr[...] = kth
    anyhit = jnp.any(d2 <= t10, axis=0, keepdims=True)
    hits_scr[...] = jnp.where(anyhit, 0.5, 0.0)

    @pl.when(bad)
    def _fallback():
        mm = jnp.min(d2, axis=1, keepdims=True)
        kk = mm
        for k in range(1, NK):
            mm = jnp.min(jnp.where(d2 <= mm, BIG, d2), axis=1,
                         keepdims=True)
            if k == KTH:
                kk = mm
        kth_scr[...] = kk
        ah = jnp.any(d2 <= mm, axis=0, keepdims=True)
        hits_scr[...] = jnp.where(ah, 0.5, 0.0)

    kth = kth_scr[...]
    hits = hits_scr[...]

    lr = jax.lax.broadcasted_iota(jnp.int32, (r, r), 0)
    lc = jax.lax.broadcasted_iota(jnp.int32, (r, r), 1)
    kl = jnp.sum(jnp.where(lr == lc, kth, 0.0), axis=0, keepdims=True)
    kth_lane_ref[:, pl.ds(i * r, r)] = kl

    @pl.when(i == 0)
    def _init():
        colsel_ref[...] = jnp.zeros_like(colsel_ref)

    colsel_ref[...] = jnp.maximum(colsel_ref[...], hits)

    @pl.when(i == n - 1)
    def _sigma():
        x = kth_lane_ref[...]

        def body(_, lohi):
            lo, hi = lohi
            mid = lo + ((hi - lo) >> 1)
            t = jax.lax.bitcast_convert_type(
                jnp.full((1, B), mid, jnp.int32), jnp.float32)
            cnt = jnp.sum((x <= t).astype(jnp.float32))
            return (jnp.where(cnt >= float(MED_RANK), lo, mid + 1),
                    jnp.where(cnt >= float(MED_RANK), mid, hi))

        lo, _ = jax.lax.fori_loop(
            0, 31, body, (jnp.int32(0), jnp.int32(0x7F800000)))
        med_d2 = jax.lax.bitcast_convert_type(lo, jnp.float32)
        sigma_ref[...] = jnp.broadcast_to(jnp.sqrt(jnp.sqrt(med_d2)), (1, 1))


def _aff_kernel(sigma_ref, colsel_ref, dist_ref, out_ref):
    i = pl.program_id(0)
    sigma = sigma_ref[0, 0]
    neg_inv = -1.0 / (2.0 * sigma * sigma)

    d2 = dist_ref[...].astype(jnp.float32)
    r = d2.shape[0]
    e = jnp.exp(jnp.sqrt(d2) * neg_inv)

    cs_lane = colsel_ref[...]
    cs_chunk = colsel_ref[:, pl.ds(i * r, r)]
    lr = jax.lax.broadcasted_iota(jnp.int32, (r, r), 0)
    lc = jax.lax.broadcasted_iota(jnp.int32, (r, r), 1)
    cs_row = jnp.sum(jnp.where(lr == lc, cs_chunk, 0.0), axis=1,
                     keepdims=True)
    out_ref[...] = e * (cs_row + cs_lane)


def kernel(inp):
    n1 = B // ROWS1
    dist, kth_lane, colsel, sigma = pl.pallas_call(
        _knn_kernel,
        grid=(n1,),
        in_specs=[
            pl.BlockSpec((B, D), lambda i: (0, 0)),
            pl.BlockSpec((ROWS1, D), lambda i: (i, 0)),
        ],
        out_specs=[
            pl.BlockSpec((ROWS1, B), lambda i: (i, 0)),
            pl.BlockSpec((1, B), lambda i: (0, 0)),
            pl.BlockSpec((1, B), lambda i: (0, 0)),
            pl.BlockSpec((1, 1), lambda i: (0, 0)),
        ],
        out_shape=[
            jax.ShapeDtypeStruct((B, B), jnp.bfloat16),
            jax.ShapeDtypeStruct((1, B), jnp.float32),
            jax.ShapeDtypeStruct((1, B), jnp.float32),
            jax.ShapeDtypeStruct((1, 1), jnp.float32),
        ],
        scratch_shapes=[
            pltpu.VMEM((ROWS1, 1), jnp.float32),
            pltpu.VMEM((1, B), jnp.float32),
        ],
        compiler_params=pltpu.CompilerParams(
            dimension_semantics=("arbitrary",)),
    )(inp, inp)

    n2 = B // ROWS2
    sym = pl.pallas_call(
        _aff_kernel,
        grid=(n2,),
        in_specs=[
            pl.BlockSpec((1, 1), lambda i: (0, 0)),
            pl.BlockSpec((1, B), lambda i: (0, 0)),
            pl.BlockSpec((ROWS2, B), lambda i: (i, 0)),
        ],
        out_specs=pl.BlockSpec((ROWS2, B), lambda i: (i, 0)),
        out_shape=jax.ShapeDtypeStruct((B, B), jnp.float32),
        compiler_params=pltpu.CompilerParams(
            dimension_semantics=("parallel",)),
    )(sigma, colsel, dist)
    return sym
